# initial kernel scaffold (unmeasured)
import jax
import jax.numpy as jnp
from jax import lax
from jax.experimental import pallas as pl
from jax.experimental.pallas import tpu as pltpu

N_DEV = 4


def _layer(x, Win, Wout, cid):
    b, _ = x.shape
    h_dim = Win.shape[1]
    out_dim = Wout.shape[1]

    def body(x_ref, win_ref, wout_ref, out_ref, acc_ref, comm_ref,
             send_sems, recv_sems):
        my = lax.axis_index("i")
        left = (my - 1) % N_DEV
        right = (my + 1) % N_DEV

        barrier_sem = pltpu.get_barrier_semaphore()
        for nbr in (left, right):
            pl.semaphore_signal(
                barrier_sem, inc=1,
                device_id=(nbr,), device_id_type=pl.DeviceIdType.MESH,
            )
        pl.semaphore_wait(barrier_sem, 2)

        partial = jnp.dot(x_ref[...], win_ref[...],
                          preferred_element_type=jnp.float32)
        comm_ref[0, :, :] = partial
        acc_ref[...] = partial

        for h in range(N_DEV - 1):
            rdma = pltpu.make_async_remote_copy(
                src_ref=comm_ref.at[h],
                dst_ref=comm_ref.at[h + 1],
                send_sem=send_sems.at[h],
                recv_sem=recv_sems.at[h],
                device_id=(right,),
                device_id_type=pl.DeviceIdType.MESH,
            )
            rdma.start()
            rdma.wait()
            acc_ref[...] += comm_ref[h + 1]

        hact = jnp.maximum(acc_ref[...], 0.0)
        out_ref[...] = jnp.dot(hact, wout_ref[...],
                               preferred_element_type=jnp.float32)

    return pl.pallas_call(
        body,
        out_shape=jax.ShapeDtypeStruct((b, out_dim), jnp.float32),
        in_specs=[pl.BlockSpec(memory_space=pltpu.VMEM)] * 3,
        out_specs=pl.BlockSpec(memory_space=pltpu.VMEM),
        scratch_shapes=[
            pltpu.VMEM((b, h_dim), jnp.float32),
            pltpu.VMEM((N_DEV, b, h_dim), jnp.float32),
            pltpu.SemaphoreType.DMA((N_DEV - 1,)),
            pltpu.SemaphoreType.DMA((N_DEV - 1,)),
        ],
        compiler_params=pltpu.CompilerParams(collective_id=cid),
    )(x, Win, Wout)


def kernel(x, Win0, Wout0, Win1, Wout1, Win2, Wout2):
    x = _layer(x, Win0, Wout0, 0)
    x = _layer(x, Win1, Wout1, 1)
    x = _layer(x, Win2, Wout2, 2)
    return x


# baseline (device time: 212222 ns/iter reference)
import jax
import jax.numpy as jnp
from jax import lax
from jax.experimental import pallas as pl
from jax.experimental.pallas import tpu as pltpu

N_DEV = 4


def _hidden_allreduce(x, Win, cid):
    b, _ = x.shape
    h_dim = Win.shape[1]

    def body(x_ref, win_ref, out_ref, acc_ref, comm_ref,
             send_sems, recv_sems):
        my = lax.axis_index("i")
        left = (my - 1) % N_DEV
        right = (my + 1) % N_DEV

        barrier_sem = pltpu.get_barrier_semaphore()
        for nbr in (left, right):
            pl.semaphore_signal(
                barrier_sem, inc=1,
                device_id=(nbr,), device_id_type=pl.DeviceIdType.MESH,
            )
        pl.semaphore_wait(barrier_sem, 2)

        partial = jnp.dot(x_ref[...], win_ref[...],
                          preferred_element_type=jnp.float32)
        comm_ref[0, :, :] = partial
        acc_ref[...] = partial

        for h in range(N_DEV - 1):
            rdma = pltpu.make_async_remote_copy(
                src_ref=comm_ref.at[h],
                dst_ref=comm_ref.at[h + 1],
                send_sem=send_sems.at[h],
                recv_sem=recv_sems.at[h],
                device_id=(right,),
                device_id_type=pl.DeviceIdType.MESH,
            )
            rdma.start()
            rdma.wait()
            acc_ref[...] += comm_ref[h + 1]

        out_ref[...] = jnp.maximum(acc_ref[...], 0.0)

    return pl.pallas_call(
        body,
        out_shape=jax.ShapeDtypeStruct((b, h_dim), jnp.float32),
        in_specs=[pl.BlockSpec(memory_space=pltpu.VMEM)] * 2,
        out_specs=pl.BlockSpec(memory_space=pltpu.VMEM),
        scratch_shapes=[
            pltpu.VMEM((b, h_dim), jnp.float32),
            pltpu.VMEM((N_DEV, b, h_dim), jnp.float32),
            pltpu.SemaphoreType.DMA((N_DEV - 1,)),
            pltpu.SemaphoreType.DMA((N_DEV - 1,)),
        ],
        compiler_params=pltpu.CompilerParams(
            collective_id=cid, vmem_limit_bytes=60 * 2**20),
    )(x, Win)


def _out_proj(h, Wout):
    b = h.shape[0]
    out_dim = Wout.shape[1]

    def body(h_ref, wout_ref, out_ref):
        out_ref[...] = jnp.dot(h_ref[...], wout_ref[...],
                               preferred_element_type=jnp.float32)

    return pl.pallas_call(
        body,
        out_shape=jax.ShapeDtypeStruct((b, out_dim), jnp.float32),
        in_specs=[pl.BlockSpec(memory_space=pltpu.VMEM)] * 2,
        out_specs=pl.BlockSpec(memory_space=pltpu.VMEM),
        compiler_params=pltpu.CompilerParams(vmem_limit_bytes=60 * 2**20),
    )(h, Wout)


def kernel(x, Win0, Wout0, Win1, Wout1, Win2, Wout2):
    x = _out_proj(_hidden_allreduce(x, Win0, 0), Wout0)
    x = _out_proj(_hidden_allreduce(x, Win1, 1), Wout1)
    x = _out_proj(_hidden_allreduce(x, Win2, 2), Wout2)
    return x


# device time: 104414 ns/iter; 2.0325x vs baseline; 2.0325x over previous
import jax
import jax.numpy as jnp
from jax import lax
from jax.experimental import pallas as pl
from jax.experimental.pallas import tpu as pltpu

N_DEV = 4
F32 = jnp.float32


def kernel(x, Win0, Wout0, Win1, Wout1, Win2, Wout2):
    b, d_sh = x.shape
    h_dim = Win0.shape[1]
    C = h_dim // N_DEV
    out_dim = Wout0.shape[1]

    def body(x_ref, win0, wout0, win1, wout1, win2, wout2, out_ref,
             xa, xb, win_tile, wout_tile, rs_send, rs_recv, ag_buf, hchunk,
             win_sems, wout_sems, rs_s_sems, rs_r_sems, ag_s_sems, ag_r_sems):
        my = lax.axis_index("i")
        left = (my + 3) % N_DEV
        right = (my + 1) % N_DEV

        barrier_sem = pltpu.get_barrier_semaphore()
        for nbr in (left, right):
            pl.semaphore_signal(
                barrier_sem, inc=1,
                device_id=(nbr,), device_id_type=pl.DeviceIdType.MESH,
            )
        pl.semaphore_wait(barrier_sem, 2)

        def win_dma(win_ref, c, slot):
            return pltpu.make_async_copy(
                win_ref.at[:, pl.ds(c * C, C)], win_tile.at[slot],
                win_sems.at[slot])

        def wout_dma(wout_ref, c, slot):
            return pltpu.make_async_copy(
                wout_ref.at[pl.ds(c * C, C), :], wout_tile.at[slot],
                wout_sems.at[slot])

        def rs_rdma(s):
            return pltpu.make_async_remote_copy(
                src_ref=rs_send.at[s], dst_ref=rs_recv.at[s],
                send_sem=rs_s_sems.at[s], recv_sem=rs_r_sems.at[s],
                device_id=(right,), device_id_type=pl.DeviceIdType.MESH)

        def ag_rdma(t):
            src = hchunk if t == 0 else ag_buf.at[t - 1]
            return pltpu.make_async_remote_copy(
                src_ref=src, dst_ref=ag_buf.at[t],
                send_sem=ag_s_sems.at[t], recv_sem=ag_r_sems.at[t],
                device_id=(right,), device_id_type=pl.DeviceIdType.MESH)

        def dot(a, w):
            return jnp.dot(a, w, preferred_element_type=F32)

        def layer(x_in_ref, win_ref, wout_ref, out_acc_ref):
            cs = [(my + 3) % N_DEV, (my + 2) % N_DEV, (my + 1) % N_DEV, my]
            d0 = win_dma(win_ref, cs[0], 0); d0.start()
            d1 = win_dma(win_ref, cs[1], 1); d1.start()

            x_val = x_in_ref[...]

            d0.wait()
            rs_send[0, :, :] = dot(x_val, win_tile[0])
            r0 = rs_rdma(0); r0.start()
            d2 = win_dma(win_ref, cs[2], 0); d2.start()

            d1.wait()
            p1 = dot(x_val, win_tile[1])
            r0.wait_recv()
            rs_send[1, :, :] = rs_recv[0] + p1
            r1 = rs_rdma(1); r1.start()
            d3 = win_dma(win_ref, cs[3], 1); d3.start()

            d2.wait()
            p2 = dot(x_val, win_tile[0])
            r1.wait_recv()
            rs_send[2, :, :] = rs_recv[1] + p2
            r2 = rs_rdma(2); r2.start()

            d3.wait()
            p3 = dot(x_val, win_tile[1])
            r2.wait_recv()
            hchunk[...] = jnp.maximum(rs_recv[2] + p3, 0.0)
            r0.wait_send(); r1.wait_send(); r2.wait_send()

            a0 = ag_rdma(0); a0.start()
            e0 = wout_dma(wout_ref, cs[3], 0); e0.start()
            e1 = wout_dma(wout_ref, cs[0], 1); e1.start()
            e0.wait()
            out_acc_ref[...] = dot(hchunk[...], wout_tile[0])

            a0.wait_recv()
            a1 = ag_rdma(1); a1.start()
            e2 = wout_dma(wout_ref, cs[1], 0); e2.start()
            e1.wait()
            out_acc_ref[...] += dot(ag_buf[0], wout_tile[1])

            a1.wait_recv()
            a2 = ag_rdma(2); a2.start()
            e3 = wout_dma(wout_ref, cs[2], 1); e3.start()
            e2.wait()
            out_acc_ref[...] += dot(ag_buf[1], wout_tile[0])

            a2.wait_recv()
            e3.wait()
            out_acc_ref[...] += dot(ag_buf[2], wout_tile[1])
            a0.wait_send(); a1.wait_send(); a2.wait_send()

        layer(x_ref, win0, wout0, xa)
        layer(xa, win1, wout1, xb)
        layer(xb, win2, wout2, out_ref)

    return pl.pallas_call(
        body,
        out_shape=jax.ShapeDtypeStruct((b, out_dim), F32),
        in_specs=[pl.BlockSpec(memory_space=pltpu.VMEM)]
        + [pl.BlockSpec(memory_space=pltpu.MemorySpace.HBM)] * 6,
        out_specs=pl.BlockSpec(memory_space=pltpu.VMEM),
        scratch_shapes=[
            pltpu.VMEM((b, out_dim), F32),
            pltpu.VMEM((b, out_dim), F32),
            pltpu.VMEM((2, d_sh, C), F32),
            pltpu.VMEM((2, C, out_dim), F32),
            pltpu.VMEM((N_DEV - 1, b, C), F32),
            pltpu.VMEM((N_DEV - 1, b, C), F32),
            pltpu.VMEM((N_DEV - 1, b, C), F32),
            pltpu.VMEM((b, C), F32),
            pltpu.SemaphoreType.DMA((2,)),
            pltpu.SemaphoreType.DMA((2,)),
            pltpu.SemaphoreType.DMA((N_DEV - 1,)),
            pltpu.SemaphoreType.DMA((N_DEV - 1,)),
            pltpu.SemaphoreType.DMA((N_DEV - 1,)),
            pltpu.SemaphoreType.DMA((N_DEV - 1,)),
        ],
        compiler_params=pltpu.CompilerParams(
            collective_id=0, vmem_limit_bytes=60 * 2**20),
    )(x, Win0, Wout0, Win1, Wout1, Win2, Wout2)


# device time: 98539 ns/iter; 2.1537x vs baseline; 1.0596x over previous
import jax
import jax.numpy as jnp
from jax import lax
from jax.experimental import pallas as pl
from jax.experimental.pallas import tpu as pltpu

N_DEV = 4
F32 = jnp.float32


def kernel(x, Win0, Wout0, Win1, Wout1, Win2, Wout2):
    b, d_sh = x.shape
    h_dim = Win0.shape[1]
    C = h_dim // N_DEV
    out_dim = Wout0.shape[1]

    def body(x_ref, win0, wout0, win1, wout1, win2, wout2, out_ref,
             xa, xb, win_tile, wout_tile, rs_send, rs_recv, ag_buf, hchunk,
             win_sems, wout_sems, rs_s_sems, rs_r_sems, ag_s_sems, ag_r_sems):
        my = lax.axis_index("i")
        left = (my + 3) % N_DEV
        right = (my + 1) % N_DEV

        def win_dma(win_ref, c, slot):
            return pltpu.make_async_copy(
                win_ref.at[:, pl.ds(c * C, C)], win_tile.at[slot],
                win_sems.at[slot])

        def wout_dma(wout_ref, c, slot):
            return pltpu.make_async_copy(
                wout_ref.at[pl.ds(c * C, C), :], wout_tile.at[slot],
                wout_sems.at[slot])

        def rs_rdma(s):
            return pltpu.make_async_remote_copy(
                src_ref=rs_send.at[s], dst_ref=rs_recv.at[s],
                send_sem=rs_s_sems.at[s], recv_sem=rs_r_sems.at[s],
                device_id=(right,), device_id_type=pl.DeviceIdType.MESH)

        def ag_rdma(t):
            src = hchunk if t == 0 else ag_buf.at[t - 1]
            return pltpu.make_async_remote_copy(
                src_ref=src, dst_ref=ag_buf.at[t],
                send_sem=ag_s_sems.at[t], recv_sem=ag_r_sems.at[t],
                device_id=(right,), device_id_type=pl.DeviceIdType.MESH)

        def dot(a, w):
            return jnp.dot(a, w, preferred_element_type=F32)

        cs = [(my + 3) % N_DEV, (my + 2) % N_DEV, (my + 1) % N_DEV, my]

        def layer(x_in_ref, win_ref, wout_ref, out_acc_ref, d0, d1,
                  next_win_ref):
            x_val = x_in_ref[...]

            d0.wait()
            rs_send[0, :, :] = dot(x_val, win_tile[0])
            r0 = rs_rdma(0); r0.start()
            d2 = win_dma(win_ref, cs[2], 0); d2.start()

            d1.wait()
            p1 = dot(x_val, win_tile[1])
            r0.wait_recv()
            rs_send[1, :, :] = rs_recv[0] + p1
            r1 = rs_rdma(1); r1.start()
            d3 = win_dma(win_ref, cs[3], 1); d3.start()
            e0 = wout_dma(wout_ref, cs[3], 0); e0.start()
            e1 = wout_dma(wout_ref, cs[0], 1); e1.start()

            d2.wait()
            p2 = dot(x_val, win_tile[0])
            r1.wait_recv()
            rs_send[2, :, :] = rs_recv[1] + p2
            r2 = rs_rdma(2); r2.start()

            d3.wait()
            p3 = dot(x_val, win_tile[1])
            r2.wait_recv()
            hchunk[...] = jnp.maximum(rs_recv[2] + p3, 0.0)
            r0.wait_send(); r1.wait_send(); r2.wait_send()

            a0 = ag_rdma(0); a0.start()
            e0.wait()
            acc = dot(hchunk[...], wout_tile[0])

            a0.wait_recv()
            a1 = ag_rdma(1); a1.start()
            e2 = wout_dma(wout_ref, cs[1], 0); e2.start()
            e1.wait()
            acc += dot(ag_buf[0], wout_tile[1])

            a1.wait_recv()
            a2 = ag_rdma(2); a2.start()
            e3 = wout_dma(wout_ref, cs[2], 1); e3.start()
            nd = None
            if next_win_ref is not None:
                nd0 = win_dma(next_win_ref, cs[0], 0); nd0.start()
                nd1 = win_dma(next_win_ref, cs[1], 1); nd1.start()
                nd = (nd0, nd1)
            e2.wait()
            acc += dot(ag_buf[1], wout_tile[0])

            a2.wait_recv()
            e3.wait()
            out_acc_ref[...] = acc + dot(ag_buf[2], wout_tile[1])
            a0.wait_send(); a1.wait_send(); a2.wait_send()
            return nd

        d0 = win_dma(win0, cs[0], 0); d0.start()
        d1 = win_dma(win0, cs[1], 1); d1.start()

        barrier_sem = pltpu.get_barrier_semaphore()
        for nbr in (left, right):
            pl.semaphore_signal(
                barrier_sem, inc=1,
                device_id=(nbr,), device_id_type=pl.DeviceIdType.MESH,
            )
        pl.semaphore_wait(barrier_sem, 2)

        nd = layer(x_ref, win0, wout0, xa, d0, d1, win1)
        nd = layer(xa, win1, wout1, xb, nd[0], nd[1], win2)
        layer(xb, win2, wout2, out_ref, nd[0], nd[1], None)

    return pl.pallas_call(
        body,
        out_shape=jax.ShapeDtypeStruct((b, out_dim), F32),
        in_specs=[pl.BlockSpec(memory_space=pltpu.VMEM)]
        + [pl.BlockSpec(memory_space=pltpu.MemorySpace.HBM)] * 6,
        out_specs=pl.BlockSpec(memory_space=pltpu.VMEM),
        scratch_shapes=[
            pltpu.VMEM((b, out_dim), F32),
            pltpu.VMEM((b, out_dim), F32),
            pltpu.VMEM((2, d_sh, C), F32),
            pltpu.VMEM((2, C, out_dim), F32),
            pltpu.VMEM((N_DEV - 1, b, C), F32),
            pltpu.VMEM((N_DEV - 1, b, C), F32),
            pltpu.VMEM((N_DEV - 1, b, C), F32),
            pltpu.VMEM((b, C), F32),
            pltpu.SemaphoreType.DMA((2,)),
            pltpu.SemaphoreType.DMA((2,)),
            pltpu.SemaphoreType.DMA((N_DEV - 1,)),
            pltpu.SemaphoreType.DMA((N_DEV - 1,)),
            pltpu.SemaphoreType.DMA((N_DEV - 1,)),
            pltpu.SemaphoreType.DMA((N_DEV - 1,)),
        ],
        compiler_params=pltpu.CompilerParams(
            collective_id=0, vmem_limit_bytes=60 * 2**20),
    )(x, Win0, Wout0, Win1, Wout1, Win2, Wout2)


# device time: 91351 ns/iter; 2.3231x vs baseline; 1.0787x over previous
import jax
import jax.numpy as jnp
from jax import lax
from jax.experimental import pallas as pl
from jax.experimental.pallas import tpu as pltpu

N_DEV = 4
F32 = jnp.float32


def kernel(x, Win0, Wout0, Win1, Wout1, Win2, Wout2):
    b, d_sh = x.shape
    h_dim = Win0.shape[1]
    C = h_dim // N_DEV
    out_dim = Wout0.shape[1]

    def body(x_ref, win0, wout0, win1, wout1, win2, wout2, out_ref,
             xa, xb, win_tile, wout_tile, sbuf, rs_in, ag_in, hchunk,
             win_sems, wout_sems, rs_s_sems, rs_r_sems, ag_s_sems, ag_r_sems):
        my = lax.axis_index("i")
        peers = [(my + d) % N_DEV for d in (1, 2, 3)]

        def win_dma(win_ref, c, slot):
            return pltpu.make_async_copy(
                win_ref.at[:, pl.ds(c * C, C)], win_tile.at[slot],
                win_sems.at[slot])

        def wout_dma(wout_ref, c, slot):
            return pltpu.make_async_copy(
                wout_ref.at[pl.ds(c * C, C), :], wout_tile.at[slot],
                wout_sems.at[slot])

        def rs_rdma(k):
            return pltpu.make_async_remote_copy(
                src_ref=sbuf.at[k], dst_ref=rs_in.at[k],
                send_sem=rs_s_sems.at[k], recv_sem=rs_r_sems.at[k],
                device_id=(peers[k],), device_id_type=pl.DeviceIdType.MESH)

        def ag_rdma(k):
            return pltpu.make_async_remote_copy(
                src_ref=hchunk, dst_ref=ag_in.at[k],
                send_sem=ag_s_sems.at[k], recv_sem=ag_r_sems.at[k],
                device_id=(peers[k],), device_id_type=pl.DeviceIdType.MESH)

        def dot(a, w):
            return jnp.dot(a, w, preferred_element_type=F32)

        win_order = peers + [my]
        wout_order = [my, (my + 3) % N_DEV, (my + 1) % N_DEV, (my + 2) % N_DEV]
        ag_slot_order = [0, 2, 1]

        def layer(x_in_ref, win_ref, wout_ref, out_acc_ref, d0, d1,
                  next_win_ref):
            x_val = x_in_ref[...]

            d0.wait()
            sbuf[0, :, :] = dot(x_val, win_tile[0])
            r0 = rs_rdma(0); r0.start()
            d2 = win_dma(win_ref, win_order[2], 0); d2.start()

            d1.wait()
            sbuf[1, :, :] = dot(x_val, win_tile[1])
            r1 = rs_rdma(1); r1.start()
            d3 = win_dma(win_ref, win_order[3], 1); d3.start()

            d2.wait()
            sbuf[2, :, :] = dot(x_val, win_tile[0])
            r2 = rs_rdma(2); r2.start()
            e0 = wout_dma(wout_ref, wout_order[0], 0); e0.start()
            e1 = wout_dma(wout_ref, wout_order[1], 1); e1.start()

            d3.wait()
            p_own = dot(x_val, win_tile[1])
            r0.wait_recv(); r1.wait_recv(); r2.wait_recv()
            hchunk[...] = jnp.maximum(
                p_own + rs_in[0] + rs_in[1] + rs_in[2], 0.0)
            r0.wait_send(); r1.wait_send(); r2.wait_send()

            a0 = ag_rdma(0); a0.start()
            a1 = ag_rdma(1); a1.start()
            a2 = ag_rdma(2); a2.start()
            ags = [a0, a1, a2]

            e0.wait()
            acc = dot(hchunk[...], wout_tile[0])
            e2 = wout_dma(wout_ref, wout_order[2], 0); e2.start()

            ags[ag_slot_order[0]].wait_recv()
            e1.wait()
            acc += dot(ag_in[ag_slot_order[0]], wout_tile[1])
            e3 = wout_dma(wout_ref, wout_order[3], 1); e3.start()
            nd = None
            if next_win_ref is not None:
                nd0 = win_dma(next_win_ref, win_order[0], 0); nd0.start()
                nd1 = win_dma(next_win_ref, win_order[1], 1); nd1.start()
                nd = (nd0, nd1)

            ags[ag_slot_order[1]].wait_recv()
            e2.wait()
            acc += dot(ag_in[ag_slot_order[1]], wout_tile[0])

            ags[ag_slot_order[2]].wait_recv()
            e3.wait()
            out_acc_ref[...] = acc + dot(ag_in[ag_slot_order[2]], wout_tile[1])
            a0.wait_send(); a1.wait_send(); a2.wait_send()
            return nd

        d0 = win_dma(win0, win_order[0], 0); d0.start()
        d1 = win_dma(win0, win_order[1], 1); d1.start()

        barrier_sem = pltpu.get_barrier_semaphore()
        for p in peers:
            pl.semaphore_signal(
                barrier_sem, inc=1,
                device_id=(p,), device_id_type=pl.DeviceIdType.MESH,
            )
        pl.semaphore_wait(barrier_sem, 3)

        nd = layer(x_ref, win0, wout0, xa, d0, d1, win1)
        nd = layer(xa, win1, wout1, xb, nd[0], nd[1], win2)
        layer(xb, win2, wout2, out_ref, nd[0], nd[1], None)

    return pl.pallas_call(
        body,
        out_shape=jax.ShapeDtypeStruct((b, out_dim), F32),
        in_specs=[pl.BlockSpec(memory_space=pltpu.VMEM)]
        + [pl.BlockSpec(memory_space=pltpu.MemorySpace.HBM)] * 6,
        out_specs=pl.BlockSpec(memory_space=pltpu.VMEM),
        scratch_shapes=[
            pltpu.VMEM((b, out_dim), F32),
            pltpu.VMEM((b, out_dim), F32),
            pltpu.VMEM((2, d_sh, C), F32),
            pltpu.VMEM((2, C, out_dim), F32),
            pltpu.VMEM((N_DEV - 1, b, C), F32),
            pltpu.VMEM((N_DEV - 1, b, C), F32),
            pltpu.VMEM((N_DEV - 1, b, C), F32),
            pltpu.VMEM((b, C), F32),
            pltpu.SemaphoreType.DMA((2,)),
            pltpu.SemaphoreType.DMA((2,)),
            pltpu.SemaphoreType.DMA((N_DEV - 1,)),
            pltpu.SemaphoreType.DMA((N_DEV - 1,)),
            pltpu.SemaphoreType.DMA((N_DEV - 1,)),
            pltpu.SemaphoreType.DMA((N_DEV - 1,)),
        ],
        compiler_params=pltpu.CompilerParams(
            collective_id=0, vmem_limit_bytes=60 * 2**20),
    )(x, Win0, Wout0, Win1, Wout1, Win2, Wout2)


# device time: 72606 ns/iter; 2.9229x vs baseline; 1.2582x over previous
import jax
import jax.numpy as jnp
from jax import lax
from jax.experimental import pallas as pl
from jax.experimental.pallas import tpu as pltpu

N_DEV = 4
F32 = jnp.float32
BF16 = jnp.bfloat16


def kernel(x, Win0, Wout0, Win1, Wout1, Win2, Wout2):
    b, d_sh = x.shape
    h_dim = Win0.shape[1]
    C = h_dim // N_DEV
    out_dim = Wout0.shape[1]

    def body(x_ref, win0, wout0, win1, wout1, win2, wout2, out_ref,
             xa, xb, win_tile, wout_tile, sbuf, rs_in, ag_in, hchunk,
             win_sems, wout_sems, rs_s_sems, rs_r_sems, ag_s_sems, ag_r_sems):
        my = lax.axis_index("i")
        peers = [(my + d) % N_DEV for d in (1, 2, 3)]

        def win_dma(win_ref, c, slot):
            return pltpu.make_async_copy(
                win_ref.at[:, pl.ds(c * C, C)], win_tile.at[slot],
                win_sems.at[slot])

        def wout_dma(wout_ref, c, slot):
            return pltpu.make_async_copy(
                wout_ref.at[pl.ds(c * C, C), :], wout_tile.at[slot],
                wout_sems.at[slot])

        def rs_rdma(k):
            return pltpu.make_async_remote_copy(
                src_ref=sbuf.at[k], dst_ref=rs_in.at[k],
                send_sem=rs_s_sems.at[k], recv_sem=rs_r_sems.at[k],
                device_id=(peers[k],), device_id_type=pl.DeviceIdType.MESH)

        def ag_rdma(k):
            return pltpu.make_async_remote_copy(
                src_ref=hchunk, dst_ref=ag_in.at[k],
                send_sem=ag_s_sems.at[k], recv_sem=ag_r_sems.at[k],
                device_id=(peers[k],), device_id_type=pl.DeviceIdType.MESH)

        def dot(a, w):
            return jnp.dot(a, w, preferred_element_type=F32)

        win_order = peers + [my]
        wout_order = [my, (my + 3) % N_DEV, (my + 1) % N_DEV, (my + 2) % N_DEV]
        ag_slot_order = [0, 2, 1]

        def layer(x_in_ref, win_ref, wout_ref, out_acc_ref, d0, d1,
                  next_win_ref):
            x_val = x_in_ref[...]

            d0.wait()
            sbuf[0, :, :] = dot(x_val, win_tile[0]).astype(BF16)
            r0 = rs_rdma(0); r0.start()
            d2 = win_dma(win_ref, win_order[2], 0); d2.start()

            d1.wait()
            sbuf[1, :, :] = dot(x_val, win_tile[1]).astype(BF16)
            r1 = rs_rdma(1); r1.start()
            d3 = win_dma(win_ref, win_order[3], 1); d3.start()

            d2.wait()
            sbuf[2, :, :] = dot(x_val, win_tile[0]).astype(BF16)
            r2 = rs_rdma(2); r2.start()
            e0 = wout_dma(wout_ref, wout_order[0], 0); e0.start()
            e1 = wout_dma(wout_ref, wout_order[1], 1); e1.start()

            d3.wait()
            p_own = dot(x_val, win_tile[1])
            r0.wait_recv(); r1.wait_recv(); r2.wait_recv()
            h_own = jnp.maximum(
                p_own + (rs_in[0].astype(F32) + rs_in[1].astype(F32)
                         + rs_in[2].astype(F32)), 0.0)
            hchunk[...] = h_own.astype(BF16)
            r0.wait_send(); r1.wait_send(); r2.wait_send()

            a0 = ag_rdma(0); a0.start()
            a1 = ag_rdma(1); a1.start()
            a2 = ag_rdma(2); a2.start()
            ags = [a0, a1, a2]

            e0.wait()
            acc = dot(h_own, wout_tile[0])
            e2 = wout_dma(wout_ref, wout_order[2], 0); e2.start()

            ags[ag_slot_order[0]].wait_recv()
            e1.wait()
            acc += dot(ag_in[ag_slot_order[0]].astype(F32), wout_tile[1])
            e3 = wout_dma(wout_ref, wout_order[3], 1); e3.start()
            nd = None
            if next_win_ref is not None:
                nd0 = win_dma(next_win_ref, win_order[0], 0); nd0.start()
                nd1 = win_dma(next_win_ref, win_order[1], 1); nd1.start()
                nd = (nd0, nd1)

            ags[ag_slot_order[1]].wait_recv()
            e2.wait()
            acc += dot(ag_in[ag_slot_order[1]].astype(F32), wout_tile[0])

            ags[ag_slot_order[2]].wait_recv()
            e3.wait()
            out_acc_ref[...] = acc + dot(ag_in[ag_slot_order[2]].astype(F32), wout_tile[1])
            a0.wait_send(); a1.wait_send(); a2.wait_send()
            return nd

        d0 = win_dma(win0, win_order[0], 0); d0.start()
        d1 = win_dma(win0, win_order[1], 1); d1.start()

        barrier_sem = pltpu.get_barrier_semaphore()
        for p in peers:
            pl.semaphore_signal(
                barrier_sem, inc=1,
                device_id=(p,), device_id_type=pl.DeviceIdType.MESH,
            )
        pl.semaphore_wait(barrier_sem, 3)

        nd = layer(x_ref, win0, wout0, xa, d0, d1, win1)
        nd = layer(xa, win1, wout1, xb, nd[0], nd[1], win2)
        layer(xb, win2, wout2, out_ref, nd[0], nd[1], None)

    return pl.pallas_call(
        body,
        out_shape=jax.ShapeDtypeStruct((b, out_dim), F32),
        in_specs=[pl.BlockSpec(memory_space=pltpu.VMEM)]
        + [pl.BlockSpec(memory_space=pltpu.MemorySpace.HBM)] * 6,
        out_specs=pl.BlockSpec(memory_space=pltpu.VMEM),
        scratch_shapes=[
            pltpu.VMEM((b, out_dim), F32),
            pltpu.VMEM((b, out_dim), F32),
            pltpu.VMEM((2, d_sh, C), F32),
            pltpu.VMEM((2, C, out_dim), F32),
            pltpu.VMEM((N_DEV - 1, b, C), BF16),
            pltpu.VMEM((N_DEV - 1, b, C), BF16),
            pltpu.VMEM((N_DEV - 1, b, C), BF16),
            pltpu.VMEM((b, C), BF16),
            pltpu.SemaphoreType.DMA((2,)),
            pltpu.SemaphoreType.DMA((2,)),
            pltpu.SemaphoreType.DMA((N_DEV - 1,)),
            pltpu.SemaphoreType.DMA((N_DEV - 1,)),
            pltpu.SemaphoreType.DMA((N_DEV - 1,)),
            pltpu.SemaphoreType.DMA((N_DEV - 1,)),
        ],
        compiler_params=pltpu.CompilerParams(
            collective_id=0, vmem_limit_bytes=60 * 2**20),
    )(x, Win0, Wout0, Win1, Wout1, Win2, Wout2)


# device time: 69076 ns/iter; 3.0723x vs baseline; 1.0511x over previous
import jax
import jax.numpy as jnp
from jax import lax
from jax.experimental import pallas as pl
from jax.experimental.pallas import tpu as pltpu

N_DEV = 4
F32 = jnp.float32
BF16 = jnp.bfloat16


def kernel(x, Win0, Wout0, Win1, Wout1, Win2, Wout2):
    b, d_sh = x.shape
    h_dim = Win0.shape[1]
    C = h_dim // N_DEV
    out_dim = Wout0.shape[1]

    def body(x_ref, win0, wout0, win1, wout1, win2, wout2, out_ref,
             xa, xb, win_tile, wout_tile, sbuf, rs_in, ag_in, hchunk,
             win_sems, wout_sems, rs_s_sems, rs_r_sems, ag_s_sems, ag_r_sems):
        my = lax.axis_index("i")
        peers = [(my + d) % N_DEV for d in (1, 2, 3)]

        def win_dma(win_ref, c, slot):
            return pltpu.make_async_copy(
                win_ref.at[:, pl.ds(c * C, C)], win_tile.at[slot],
                win_sems.at[slot])

        def wout_dma(wout_ref, c, slot):
            return pltpu.make_async_copy(
                wout_ref.at[pl.ds(c * C, C), :], wout_tile.at[slot],
                wout_sems.at[slot])

        def rs_rdma(k):
            return pltpu.make_async_remote_copy(
                src_ref=sbuf.at[k], dst_ref=rs_in.at[k],
                send_sem=rs_s_sems.at[k], recv_sem=rs_r_sems.at[k],
                device_id=(peers[k],), device_id_type=pl.DeviceIdType.MESH)

        def ag_rdma(k):
            return pltpu.make_async_remote_copy(
                src_ref=hchunk, dst_ref=ag_in.at[k],
                send_sem=ag_s_sems.at[k], recv_sem=ag_r_sems.at[k],
                device_id=(peers[k],), device_id_type=pl.DeviceIdType.MESH)

        def dot(a, w):
            return jnp.dot(a, w, preferred_element_type=F32)

        win_order = peers + [my]
        wout_order = [my, (my + 3) % N_DEV, (my + 1) % N_DEV, (my + 2) % N_DEV]
        ag_slot_order = [0, 2, 1]

        def layer(x_in_ref, win_ref, wout_ref, out_acc_ref, d0, d1,
                  next_win_ref):
            x_val = x_in_ref[...]

            d0.wait()
            sbuf[0, :, :] = dot(x_val, win_tile[0]).astype(BF16)
            r0 = rs_rdma(0); r0.start()
            d2 = win_dma(win_ref, win_order[2], 0); d2.start()

            d1.wait()
            sbuf[1, :, :] = dot(x_val, win_tile[1]).astype(BF16)
            r1 = rs_rdma(1); r1.start()
            d3 = win_dma(win_ref, win_order[3], 1); d3.start()

            d2.wait()
            sbuf[2, :, :] = dot(x_val, win_tile[0]).astype(BF16)
            r2 = rs_rdma(2); r2.start()
            e0 = wout_dma(wout_ref, wout_order[0], 0); e0.start()
            e1 = wout_dma(wout_ref, wout_order[1], 1); e1.start()
            e2 = wout_dma(wout_ref, wout_order[2], 2); e2.start()
            e3 = wout_dma(wout_ref, wout_order[3], 3); e3.start()
            nd = None
            if next_win_ref is not None:
                nd0 = win_dma(next_win_ref, win_order[0], 0); nd0.start()

            d3.wait()
            p_own = dot(x_val, win_tile[1])
            if next_win_ref is not None:
                nd1 = win_dma(next_win_ref, win_order[1], 1); nd1.start()
                nd = (nd0, nd1)
            r0.wait_recv(); r1.wait_recv(); r2.wait_recv()
            h_own = jnp.maximum(
                p_own + (rs_in[0].astype(F32) + rs_in[1].astype(F32)
                         + rs_in[2].astype(F32)), 0.0)
            hchunk[...] = h_own.astype(BF16)
            r0.wait_send(); r1.wait_send(); r2.wait_send()

            a0 = ag_rdma(0); a0.start()
            a1 = ag_rdma(1); a1.start()
            a2 = ag_rdma(2); a2.start()
            ags = [a0, a1, a2]

            e0.wait()
            acc = dot(h_own, wout_tile[0])

            ags[ag_slot_order[0]].wait_recv()
            e1.wait()
            acc += dot(ag_in[ag_slot_order[0]].astype(F32), wout_tile[1])

            ags[ag_slot_order[1]].wait_recv()
            e2.wait()
            acc += dot(ag_in[ag_slot_order[1]].astype(F32), wout_tile[2])

            ags[ag_slot_order[2]].wait_recv()
            e3.wait()
            out_acc_ref[...] = acc + dot(ag_in[ag_slot_order[2]].astype(F32), wout_tile[3])
            a0.wait_send(); a1.wait_send(); a2.wait_send()
            return nd

        d0 = win_dma(win0, win_order[0], 0); d0.start()
        d1 = win_dma(win0, win_order[1], 1); d1.start()

        barrier_sem = pltpu.get_barrier_semaphore()
        for p in peers:
            pl.semaphore_signal(
                barrier_sem, inc=1,
                device_id=(p,), device_id_type=pl.DeviceIdType.MESH,
            )
        pl.semaphore_wait(barrier_sem, 3)

        nd = layer(x_ref, win0, wout0, xa, d0, d1, win1)
        nd = layer(xa, win1, wout1, xb, nd[0], nd[1], win2)
        layer(xb, win2, wout2, out_ref, nd[0], nd[1], None)

    return pl.pallas_call(
        body,
        out_shape=jax.ShapeDtypeStruct((b, out_dim), F32),
        in_specs=[pl.BlockSpec(memory_space=pltpu.VMEM)]
        + [pl.BlockSpec(memory_space=pltpu.MemorySpace.HBM)] * 6,
        out_specs=pl.BlockSpec(memory_space=pltpu.VMEM),
        scratch_shapes=[
            pltpu.VMEM((b, out_dim), F32),
            pltpu.VMEM((b, out_dim), F32),
            pltpu.VMEM((2, d_sh, C), F32),
            pltpu.VMEM((4, C, out_dim), F32),
            pltpu.VMEM((N_DEV - 1, b, C), BF16),
            pltpu.VMEM((N_DEV - 1, b, C), BF16),
            pltpu.VMEM((N_DEV - 1, b, C), BF16),
            pltpu.VMEM((b, C), BF16),
            pltpu.SemaphoreType.DMA((2,)),
            pltpu.SemaphoreType.DMA((4,)),
            pltpu.SemaphoreType.DMA((N_DEV - 1,)),
            pltpu.SemaphoreType.DMA((N_DEV - 1,)),
            pltpu.SemaphoreType.DMA((N_DEV - 1,)),
            pltpu.SemaphoreType.DMA((N_DEV - 1,)),
        ],
        compiler_params=pltpu.CompilerParams(
            collective_id=0, vmem_limit_bytes=60 * 2**20),
    )(x, Win0, Wout0, Win1, Wout1, Win2, Wout2)
